# Initial kernel scaffold; baseline (speedup 1.0000x reference)
import jax
import jax.numpy as jnp
from jax.experimental import pallas as pl


def kernel(x, expert_indices, expert_weights, w1_stacked, w2_stacked, w3_stacked):
    raise NotImplementedError("write your pallas kernel here")


# speedup vs baseline: 4.9321x; 4.9321x over previous
import functools

import jax
import jax.numpy as jnp
from jax.experimental import pallas as pl
from jax.experimental.pallas import tpu as pltpu


def _ffn_kernel(used_ref, nu_ref, x_ref, cmat_ref, w1_ref, w2_ref, w3_ref,
                out_ref):
    i = pl.program_id(0)

    @pl.when(i == 0)
    def _init():
        out_ref[...] = jnp.zeros_like(out_ref)

    @pl.when(i < nu_ref[0])
    def _body():
        x = x_ref[...]
        g = jnp.dot(x, w1_ref[0], preferred_element_type=jnp.float32)
        u = jnp.dot(x, w3_ref[0], preferred_element_type=jnp.float32)
        h = g * jax.lax.logistic(g) * u
        y = jnp.dot(h, w2_ref[0], preferred_element_type=jnp.float32)
        c = cmat_ref[i, :]
        out_ref[...] += c[:, None] * y


def _route(ei, ew, num_experts):
    t, k = ei.shape
    n = t * k
    flat = ei.reshape(n).astype(jnp.int32)
    se = jnp.sort(flat)
    first = jnp.concatenate(
        [jnp.ones((1,), jnp.bool_), se[1:] != se[:-1]])
    nu = first.sum(dtype=jnp.int32)
    pos = jnp.cumsum(first) - 1
    used0 = jnp.zeros((n,), jnp.int32).at[pos].set(se)
    used = jnp.where(jnp.arange(n) < nu, used0, se[n - 1])
    onehot = ei[None, :, :] == jnp.arange(num_experts, dtype=jnp.int32)[:, None, None]
    c_all = (onehot * ew[None, :, :]).sum(-1)
    cmat = c_all[used] * (jnp.arange(n) < nu)[:, None]
    return used, jnp.full((1,), nu, jnp.int32), cmat


@jax.jit
def kernel(x, expert_indices, expert_weights, w1_stacked, w2_stacked,
           w3_stacked):
    t, h = x.shape
    e, _, inter = w1_stacked.shape
    k = expert_indices.shape[1]
    n = t * k

    used, nu, cmat = _route(expert_indices.astype(jnp.int32),
                            expert_weights, e)

    grid_spec = pltpu.PrefetchScalarGridSpec(
        num_scalar_prefetch=2,
        grid=(n,),
        in_specs=[
            pl.BlockSpec((t, h), lambda i, used, nu: (0, 0)),
            pl.BlockSpec((n, t), lambda i, used, nu: (0, 0)),
            pl.BlockSpec((1, h, inter), lambda i, used, nu: (used[i], 0, 0)),
            pl.BlockSpec((1, inter, h), lambda i, used, nu: (used[i], 0, 0)),
            pl.BlockSpec((1, h, inter), lambda i, used, nu: (used[i], 0, 0)),
        ],
        out_specs=pl.BlockSpec((t, h), lambda i, used, nu: (0, 0)),
    )
    return pl.pallas_call(
        _ffn_kernel,
        grid_spec=grid_spec,
        out_shape=jax.ShapeDtypeStruct((t, h), jnp.float32),
    )(used, nu, x, cmat, w1_stacked, w2_stacked, w3_stacked)
